# baseline (speedup 1.0000x reference)
import functools

import jax
import jax.numpy as jnp
from jax import lax
from jax.experimental import pallas as pl
from jax.experimental.pallas import tpu as pltpu
from jax.experimental.pallas import tpu_sc as plsc

N = 10000
NPAD = 10240
E = 320000
H = 128
G = 64
L = 4

NC = 2
NS = 16
NW = NC * NS
EPW = E // NW
BB = 80
NBLK = EPW // BB
NBLKP = 513
NBLKV = 128
ACCW = 144
RPT = NPAD // NS
TB = 512
NTB = NPAD // TB

_f32 = jnp.float32
_i32 = jnp.int32


def _widx():
    return lax.axis_index("s") * NC + lax.axis_index("c")


def _pass_a_body(xl_ref, xr_ref, srcm_ref, dstm_ref, att_ref,
                 e_ref, pmax_ref,
                 src_v, dst_v, att_v, emax_v, e_v, abuf, tmpb, xlbuf, xrbuf,
                 sem_g):
    wid = _widx()
    pltpu.sync_copy(srcm_ref.at[wid, pl.ds(0, NBLKV)], src_v)
    pltpu.sync_copy(dstm_ref.at[wid, pl.ds(0, NBLKV)], dst_v)
    pltpu.sync_copy(att_ref, att_v)

    def _init(i, carry):
        emax_v[pl.ds(i * 16, 16)] = jnp.full((16,), -1e30, _f32)
        return carry
    lax.fori_loop(0, NPAD // 16, _init, 0)

    lane = lax.iota(_i32, 16)
    att16 = [att_v[pl.ds(k * 16, 16)] for k in range(H // 16)]

    for pb in range(2):
        pltpu.async_copy(xl_ref.at[src_v.at[pb]], xlbuf.at[pb], sem_g)
        pltpu.async_copy(xr_ref.at[dst_v.at[pb]], xrbuf.at[pb], sem_g)

    def _blk(b, carry):
        p = b % 3

        @pl.when(b + 2 < NBLK)
        def _():
            pn = (b + 2) % 3
            pltpu.async_copy(xl_ref.at[src_v.at[b + 2]], xlbuf.at[pn],
                             sem_g)
            pltpu.async_copy(xr_ref.at[dst_v.at[b + 2]], xrbuf.at[pn],
                             sem_g)
        pltpu.make_async_copy(xl_ref.at[src_v.at[b]], xlbuf.at[p],
                              sem_g).wait()
        pltpu.make_async_copy(xr_ref.at[dst_v.at[b]], xrbuf.at[p],
                              sem_g).wait()
        for g in range(BB // 16):
            def _edge(u, c2):
                for i in range(4):
                    j = g * 16 + u * 4 + i
                    acc = jnp.zeros((16,), _f32)
                    for k in range(H // 16):
                        sl = pl.ds(k * 16, 16)
                        z = xlbuf[p, j, sl] + xrbuf[p, j, sl]
                        zl = jnp.where(z > 0, z, 0.2 * z)
                        acc = acc + zl * att16[k]
                    plsc.store_scatter(abuf, [lane * 16 + (u * 4 + i)], acc)
                return c2
            lax.fori_loop(0, 4, _edge, 0)
            e16 = abuf[pl.ds(0, 16)]
            for l in range(1, 16):
                e16 = e16 + abuf[pl.ds(l * 16, 16)]
            e_v[b, pl.ds(g * 16, 16)] = plsc.bitcast(e16, _i32)
            dst16 = dst_v[b, pl.ds(g * 16, 16)]

            idxm = dst16 & (2048 - 1)
            plsc.store_scatter(tmpb, [idxm], lane)
            rb = plsc.load_gather(tmpb, [idxm])
            winm = rb == lane
            cur = plsc.load_gather(emax_v, [dst16])
            plsc.store_scatter(emax_v, [dst16], jnp.maximum(cur, e16),
                               mask=winm)
            loser = jnp.logical_not(winm)

            @pl.when(jnp.max(loser.astype(_i32)) > 0)
            def _():
                def _mx(t, c2):
                    cur2 = plsc.load_gather(emax_v, [dst16])
                    need = (e16 > cur2) & (lane == t) & loser
                    plsc.store_scatter(emax_v, [dst16], e16, mask=need)
                    return c2
                lax.fori_loop(0, 16, _mx, 0)
        return carry
    lax.fori_loop(0, NBLK, _blk, 0)

    pltpu.sync_copy(e_v, e_ref.at[wid, pl.ds(0, NBLKV)])
    pltpu.sync_copy(emax_v, pmax_ref.at[wid])


def _pass_a(xl, xr, srcm, dstm, att_l):
    mesh = plsc.VectorSubcoreMesh(core_axis_name="c", subcore_axis_name="s")
    f = pl.kernel(
        _pass_a_body,
        out_type=(jax.ShapeDtypeStruct((NW, NBLKP, BB), _i32),
                  jax.ShapeDtypeStruct((NW, NPAD), _f32)),
        mesh=mesh,
        compiler_params=pltpu.CompilerParams(needs_layout_passes=False),
        scratch_types=[
            pltpu.VMEM((NBLKV, BB), _i32),
            pltpu.VMEM((NBLKV, BB), _i32),
            pltpu.VMEM((H,), _f32),
            pltpu.VMEM((NPAD,), _f32),
            pltpu.VMEM((NBLKV, BB), _i32),
            pltpu.VMEM((256,), _f32),
            pltpu.VMEM((2048,), _i32),
            pltpu.VMEM((3, BB, H), _f32),
            pltpu.VMEM((3, BB, H), _f32),
            pltpu.SemaphoreType.DMA,
        ],
    )
    return f(xl, xr, srcm, dstm, att_l)


def _pass_b_body(e_ref, pmax_ref, xl_ref, srcm_ref, dstm2_ref,
                 acc_ref, denp_ref,
                 win_src, win_dst, win_e, emax_v, den_v, eebuf, xlbuf,
                 sem_g, sem_s, acc_sh):
    c = lax.axis_index("c")
    s = lax.axis_index("s")
    wid = s * NC + c

    pltpu.sync_copy(pmax_ref.at[0], emax_v)

    def _pred(p, carry):
        pltpu.sync_copy(pmax_ref.at[p], den_v)

        def _mx(i, c2):
            sl = pl.ds(i * 16, 16)
            emax_v[sl] = jnp.maximum(emax_v[sl], den_v[sl])
            return c2
        lax.fori_loop(0, NPAD // 16, _mx, 0)
        return carry
    lax.fori_loop(1, NW, _pred, 0)

    def _dz(i, carry):
        den_v[pl.ds(i * 16, 16)] = jnp.zeros((16,), _f32)
        return carry
    lax.fori_loop(0, NPAD // 16, _dz, 0)

    def _zr(r, carry):
        for j in range(H // 16):
            xlbuf[0, r, pl.ds(j * 16, 16)] = jnp.zeros((16,), _f32)
        return carry
    lax.fori_loop(0, BB, _zr, 0)
    base = s * RPT
    for i in range(RPT // BB):
        pltpu.sync_copy(xlbuf.at[0], acc_sh.at[pl.ds(base + i * BB, BB)])
    plsc.subcore_barrier()

    lane = lax.iota(_i32, 16)
    lanef = plsc.bitcast(lane, _f32)
    WW = 8
    NSUB = BB // 16

    def _wait_sub():
        pltpu.make_async_copy(xlbuf.at[0, pl.ds(0, 16)],
                              acc_sh.at[win_dst.at[0]], sem_s).wait()

    def _win(w, carry):
        pltpu.sync_copy(srcm_ref.at[wid, pl.ds(w * WW, WW)], win_src)
        pltpu.sync_copy(dstm2_ref.at[wid, pl.ds(w * WW * NSUB, WW * NSUB)],
                        win_dst)
        pltpu.sync_copy(e_ref.at[wid, pl.ds(w * WW, WW)], win_e)
        nblk_here = jnp.minimum(WW, NBLK - w * WW)
        pltpu.async_copy(xl_ref.at[win_src.at[0]], xlbuf.at[0], sem_g)

        def _blk(t, c3):
            p = t % 2

            @pl.when(w * WW + t >= 1)
            def _():
                for _ in range(NSUB):
                    _wait_sub()

            @pl.when(t + 1 < nblk_here)
            def _():
                pltpu.async_copy(xl_ref.at[win_src.at[t + 1]],
                                 xlbuf.at[1 - p], sem_g)
            for g in range(BB // 16):
                sl = pl.ds(g * 16, 16)
                dst16 = win_dst[t * NSUB + g, pl.ds(0, 16)]
                e16 = plsc.bitcast(win_e[t, sl], _f32)
                m16 = plsc.load_gather(emax_v, [dst16])
                ee16 = jnp.exp(e16 - m16)
                eebuf[sl] = ee16

                idxm = 512 + (dst16 & 511)
                plsc.store_scatter(eebuf, [idxm], lanef)
                rb = plsc.bitcast(plsc.load_gather(eebuf, [idxm]), _i32)
                winm = rb == lane
                cur = plsc.load_gather(den_v, [dst16])
                plsc.store_scatter(den_v, [dst16], cur + ee16, mask=winm)
                loser = jnp.logical_not(winm)

                @pl.when(jnp.max(loser.astype(_i32)) > 0)
                def _():
                    def _acc(u, c4):
                        cur2 = plsc.load_gather(den_v, [dst16])
                        plsc.store_scatter(den_v, [dst16], cur2 + ee16,
                                           mask=(lane == u) & loser)
                        return c4
                    lax.fori_loop(0, 16, _acc, 0)

            pltpu.make_async_copy(xl_ref.at[win_src.at[t]], xlbuf.at[p],
                                  sem_g).wait()

            for g in range(NSUB):
                def _edge(u, c4):
                    for i in range(4):
                        j = g * 16 + u * 4 + i
                        spl = plsc.load_gather(eebuf,
                                               [jnp.full((16,), j, _i32)])
                        for k in range(H // 16):
                            sl2 = pl.ds(k * 16, 16)
                            xlbuf[p, j, sl2] = xlbuf[p, j, sl2] * spl
                    return c4
                lax.fori_loop(0, 4, _edge, 0)
                pltpu.async_copy(xlbuf.at[p, pl.ds(g * 16, 16)],
                                 acc_sh.at[win_dst.at[t * NSUB + g]],
                                 sem_s, add=True)
            return c3
        lax.fori_loop(0, nblk_here, _blk, 0)
        return carry
    lax.fori_loop(0, (NBLK + WW - 1) // WW, _win, 0)
    for _ in range(NSUB):
        _wait_sub()
    pltpu.sync_copy(den_v, denp_ref.at[wid])
    plsc.subcore_barrier()

    for i in range(RPT // BB):
        sl = pl.ds(base + i * BB, BB)
        pltpu.sync_copy(acc_sh.at[sl], acc_ref.at[c, sl])


def _pass_b(e, pmax, xl, srcm, dstm2):
    mesh = plsc.VectorSubcoreMesh(core_axis_name="c", subcore_axis_name="s")
    f = pl.kernel(
        _pass_b_body,
        out_type=(jax.ShapeDtypeStruct((NC, NPAD, H), _f32),
                  jax.ShapeDtypeStruct((NW, NPAD), _f32)),
        mesh=mesh,
        compiler_params=pltpu.CompilerParams(needs_layout_passes=False),
        scratch_types=[
            pltpu.VMEM((8, BB), _i32),
            pltpu.VMEM((8 * (BB // 16), 16), _i32),
            pltpu.VMEM((8, BB), _i32),
            pltpu.VMEM((NPAD,), _f32),
            pltpu.VMEM((NPAD,), _f32),
            pltpu.VMEM((1024,), _f32),
            pltpu.VMEM((2, BB, H), _f32),
            pltpu.SemaphoreType.DMA,
            pltpu.SemaphoreType.DMA,
            pltpu.VMEM_SHARED((NPAD, H), _f32),
        ],
    )
    return f(e, pmax, xl, srcm, dstm2)


def _mm0_body(x_ref, wl_ref, wr_ref, xl_ref, xr_ref):
    h = x_ref[...]
    xl_ref[...] = jnp.dot(h, wl_ref[...], preferred_element_type=_f32)
    xr_ref[...] = jnp.dot(h, wr_ref[...], preferred_element_type=_f32)


def _mm0(x, wl, wr):
    return pl.pallas_call(
        _mm0_body,
        grid=(NTB,),
        in_specs=[
            pl.BlockSpec((TB, H), lambda i: (i, 0)),
            pl.BlockSpec((H, H), lambda i: (0, 0)),
            pl.BlockSpec((H, H), lambda i: (0, 0)),
        ],
        out_specs=[
            pl.BlockSpec((TB, H), lambda i: (i, 0)),
            pl.BlockSpec((TB, H), lambda i: (i, 0)),
        ],
        out_shape=[
            jax.ShapeDtypeStruct((NPAD, H), _f32),
            jax.ShapeDtypeStruct((NPAD, H), _f32),
        ],
    )(x, wl, wr)


def _combine_h(acc_blk, denp_blk, bias_row):
    den = lax.dot_general(denp_blk, jnp.ones((NW, 1), _f32),
                          (((0,), (0,)), ((), ())),
                          preferred_element_type=_f32)
    return jnp.maximum((acc_blk[0] + acc_blk[1]) / (den + 1e-16)
                       + bias_row, 0.0)


def _comb_body(acc_ref, denp_ref, bias_ref, wl_ref, wr_ref,
               xl_ref, xr_ref):
    h = _combine_h(acc_ref[...], denp_ref[...], bias_ref[...])
    xl_ref[...] = jnp.dot(h, wl_ref[...], preferred_element_type=_f32)
    xr_ref[...] = jnp.dot(h, wr_ref[...], preferred_element_type=_f32)


def _combine_mm(acc, denp, bias_row, wl, wr):
    return pl.pallas_call(
        _comb_body,
        grid=(NTB,),
        in_specs=[
            pl.BlockSpec((NC, TB, H), lambda i: (0, i, 0)),
            pl.BlockSpec((NW, TB), lambda i: (0, i)),
            pl.BlockSpec((1, H), lambda i: (0, 0)),
            pl.BlockSpec((H, H), lambda i: (0, 0)),
            pl.BlockSpec((H, H), lambda i: (0, 0)),
        ],
        out_specs=[
            pl.BlockSpec((TB, H), lambda i: (i, 0)),
            pl.BlockSpec((TB, H), lambda i: (i, 0)),
        ],
        out_shape=[
            jax.ShapeDtypeStruct((NPAD, H), _f32),
            jax.ShapeDtypeStruct((NPAD, H), _f32),
        ],
    )(acc, denp, bias_row, wl, wr)


def _fin_body(acc_ref, denp_ref, bias_ref, batch_ref, w1_ref, b1_ref,
              w2_ref, b2_ref, out_ref, pooled_s, counts_s):
    i = pl.program_id(0)

    @pl.when(i == 0)
    def _():
        pooled_s[...] = jnp.zeros((G, H), _f32)
        counts_s[...] = jnp.zeros((G, H), _f32)

    h = _combine_h(acc_ref[...], denp_ref[...], bias_ref[...])
    bt = batch_ref[0]
    gi = lax.broadcasted_iota(_i32, (G, TB), 0).astype(_f32)
    onehot = jnp.where(bt == gi, 1.0, 0.0)
    pooled_s[...] += jnp.dot(onehot, h, preferred_element_type=_f32)
    cnt = jnp.sum(onehot, axis=1, keepdims=True)
    counts_s[...] += jnp.broadcast_to(cnt, (G, H))

    @pl.when(i == NTB - 1)
    def _():
        pooled = pooled_s[...] / jnp.maximum(counts_s[...], 1.0)
        t = jnp.dot(pooled, w1_ref[...], preferred_element_type=_f32)
        t = jnp.maximum(t + b1_ref[...], 0.0)
        out_ref[...] = (jnp.dot(t, w2_ref[...], preferred_element_type=_f32)
                        + b2_ref[...])


def _final(acc, denp, bias_row, batch3, fc1_W, fc1_b, fc2_W, fc2_b):
    return pl.pallas_call(
        _fin_body,
        grid=(NTB,),
        in_specs=[
            pl.BlockSpec((NC, TB, H), lambda i: (0, i, 0)),
            pl.BlockSpec((NW, TB), lambda i: (0, i)),
            pl.BlockSpec((1, H), lambda i: (0, 0)),
            pl.BlockSpec((1, 1, TB), lambda i: (i, 0, 0)),
            pl.BlockSpec((H, H), lambda i: (0, 0)),
            pl.BlockSpec((1, H), lambda i: (0, 0)),
            pl.BlockSpec((H, H), lambda i: (0, 0)),
            pl.BlockSpec((1, H), lambda i: (0, 0)),
        ],
        out_specs=pl.BlockSpec((G, H), lambda i: (0, 0)),
        out_shape=jax.ShapeDtypeStruct((G, H), _f32),
        scratch_shapes=[
            pltpu.VMEM((G, H), _f32),
            pltpu.VMEM((G, H), _f32),
        ],
    )(acc, denp, bias_row, batch3, fc1_W, fc1_b, fc2_W, fc2_b)


def kernel(x, edge_index, batch, Wl, Wr, att, bias, fc1_W, fc1_b,
           fc2_W, fc2_b):
    x_p = jnp.pad(x, ((0, NPAD - N), (0, 0)))
    ei = edge_index.astype(_i32).reshape(2, NW, NBLK, BB)
    ei = jnp.pad(ei, ((0, 0), (0, 0), (0, NBLKP - NBLK), (0, 0)))
    srcm, dstm = ei[0], ei[1]
    dstm2 = dstm.reshape(NW, NBLKP * (BB // 16), 16)
    batch_p = jnp.concatenate(
        [batch.astype(_i32), jnp.full((NPAD - N,), G, _i32)]).astype(_f32)
    batch3 = batch_p.reshape(NTB, 1, TB)

    xl, xr = _mm0(x_p, Wl[0], Wr[0])
    acc = denp = None
    for l in range(L):
        if l > 0:
            xl, xr = _combine_mm(acc, denp, bias[l - 1][None],
                                 Wl[l], Wr[l])
        e, pmax = _pass_a(xl, xr, srcm, dstm, att[l])
        acc, denp = _pass_b(e, pmax, xl, srcm, dstm2)
    return _final(acc, denp, bias[L - 1][None], batch3,
                  fc1_W, fc1_b[None], fc2_W, fc2_b[None])


# speedup vs baseline: 1.0036x; 1.0036x over previous
import functools

import jax
import jax.numpy as jnp
from jax import lax
from jax.experimental import pallas as pl
from jax.experimental.pallas import tpu as pltpu
from jax.experimental.pallas import tpu_sc as plsc

N = 10000
NPAD = 10240
E = 320000
H = 128
G = 64
L = 4

NC = 2
NS = 16
NW = NC * NS
EPW = E // NW
BB = 80
NBLK = EPW // BB
NBLKP = 513
NBLKV = 128
ACCW = 144
RPT = NPAD // NS
TB = 512
NTB = NPAD // TB

_f32 = jnp.float32
_i32 = jnp.int32


def _widx():
    return lax.axis_index("s") * NC + lax.axis_index("c")


def _pass_a_body(xl_ref, xr_ref, srcm_ref, dstm_ref, att_ref,
                 e_ref, pmax_ref,
                 src_v, dst_v, att_v, emax_v, e_v, abuf, tmpb, xlbuf, xrbuf,
                 sem_g):
    wid = _widx()
    pltpu.sync_copy(srcm_ref.at[wid, pl.ds(0, NBLKV)], src_v)
    pltpu.sync_copy(dstm_ref.at[wid, pl.ds(0, NBLKV)], dst_v)
    pltpu.sync_copy(att_ref, att_v)

    def _init(i, carry):
        emax_v[pl.ds(i * 16, 16)] = jnp.full((16,), -1e30, _f32)
        return carry
    lax.fori_loop(0, NPAD // 16, _init, 0)

    lane = lax.iota(_i32, 16)
    att16 = [att_v[pl.ds(k * 16, 16)] for k in range(H // 16)]

    pltpu.async_copy(xl_ref.at[src_v.at[0]], xlbuf.at[0], sem_g)
    pltpu.async_copy(xr_ref.at[dst_v.at[0]], xrbuf.at[0], sem_g)

    def _blk(b, carry):
        p = b % 2

        @pl.when(b + 1 < NBLK)
        def _():
            pltpu.async_copy(xl_ref.at[src_v.at[b + 1]], xlbuf.at[1 - p],
                             sem_g)
            pltpu.async_copy(xr_ref.at[dst_v.at[b + 1]], xrbuf.at[1 - p],
                             sem_g)
        pltpu.make_async_copy(xl_ref.at[src_v.at[b]], xlbuf.at[p],
                              sem_g).wait()
        pltpu.make_async_copy(xr_ref.at[dst_v.at[b]], xrbuf.at[p],
                              sem_g).wait()
        for g in range(BB // 16):
            def _edge(u, c2):
                for i in range(4):
                    j = g * 16 + u * 4 + i
                    acc = jnp.zeros((16,), _f32)
                    for k in range(H // 16):
                        sl = pl.ds(k * 16, 16)
                        z = xlbuf[p, j, sl] + xrbuf[p, j, sl]
                        zl = jnp.where(z > 0, z, 0.2 * z)
                        acc = acc + zl * att16[k]
                    plsc.store_scatter(abuf, [lane * 16 + (u * 4 + i)], acc)
                return c2
            lax.fori_loop(0, 4, _edge, 0)
            e16 = abuf[pl.ds(0, 16)]
            for l in range(1, 16):
                e16 = e16 + abuf[pl.ds(l * 16, 16)]
            e_v[b, pl.ds(g * 16, 16)] = plsc.bitcast(e16, _i32)
            dst16 = dst_v[b, pl.ds(g * 16, 16)]

            idxm = dst16 & (2048 - 1)
            plsc.store_scatter(tmpb, [idxm], lane)
            rb = plsc.load_gather(tmpb, [idxm])
            winm = rb == lane
            cur = plsc.load_gather(emax_v, [dst16])
            plsc.store_scatter(emax_v, [dst16], jnp.maximum(cur, e16),
                               mask=winm)
            loser = jnp.logical_not(winm)

            @pl.when(jnp.max(loser.astype(_i32)) > 0)
            def _():
                def _mx(t, c2):
                    cur2 = plsc.load_gather(emax_v, [dst16])
                    need = (e16 > cur2) & (lane == t) & loser
                    plsc.store_scatter(emax_v, [dst16], e16, mask=need)
                    return c2
                lax.fori_loop(0, 16, _mx, 0)
        return carry
    lax.fori_loop(0, NBLK, _blk, 0)

    pltpu.sync_copy(e_v, e_ref.at[wid, pl.ds(0, NBLKV)])
    pltpu.sync_copy(emax_v, pmax_ref.at[wid])


def _pass_a(xl, xr, srcm, dstm, att_l):
    mesh = plsc.VectorSubcoreMesh(core_axis_name="c", subcore_axis_name="s")
    f = pl.kernel(
        _pass_a_body,
        out_type=(jax.ShapeDtypeStruct((NW, NBLKP, BB), _i32),
                  jax.ShapeDtypeStruct((NW, NPAD), _f32)),
        mesh=mesh,
        compiler_params=pltpu.CompilerParams(needs_layout_passes=False),
        scratch_types=[
            pltpu.VMEM((NBLKV, BB), _i32),
            pltpu.VMEM((NBLKV, BB), _i32),
            pltpu.VMEM((H,), _f32),
            pltpu.VMEM((NPAD,), _f32),
            pltpu.VMEM((NBLKV, BB), _i32),
            pltpu.VMEM((256,), _f32),
            pltpu.VMEM((2048,), _i32),
            pltpu.VMEM((2, BB, H), _f32),
            pltpu.VMEM((2, BB, H), _f32),
            pltpu.SemaphoreType.DMA,
        ],
    )
    return f(xl, xr, srcm, dstm, att_l)


def _pass_b_body(e_ref, pmax_ref, xl_ref, srcm_ref, dstm2_ref,
                 acc_ref, denp_ref,
                 win_src, win_dst, win_e, emax_v, den_v, eebuf, xlbuf,
                 sem_g, sem_s, acc_sh):
    c = lax.axis_index("c")
    s = lax.axis_index("s")
    wid = s * NC + c

    pltpu.sync_copy(pmax_ref.at[0], emax_v)

    def _pred(p, carry):
        pltpu.sync_copy(pmax_ref.at[p], den_v)

        def _mx(i, c2):
            sl = pl.ds(i * 16, 16)
            emax_v[sl] = jnp.maximum(emax_v[sl], den_v[sl])
            return c2
        lax.fori_loop(0, NPAD // 16, _mx, 0)
        return carry
    lax.fori_loop(1, NW, _pred, 0)

    def _dz(i, carry):
        den_v[pl.ds(i * 16, 16)] = jnp.zeros((16,), _f32)
        return carry
    lax.fori_loop(0, NPAD // 16, _dz, 0)

    def _zr(r, carry):
        for j in range(H // 16):
            xlbuf[0, r, pl.ds(j * 16, 16)] = jnp.zeros((16,), _f32)
        return carry
    lax.fori_loop(0, BB, _zr, 0)
    base = s * RPT
    for i in range(RPT // BB):
        pltpu.sync_copy(xlbuf.at[0], acc_sh.at[pl.ds(base + i * BB, BB)])
    plsc.subcore_barrier()

    lane = lax.iota(_i32, 16)
    lanef = plsc.bitcast(lane, _f32)
    WW = 8
    NSUB = BB // 16

    def _wait_sub():
        pltpu.make_async_copy(xlbuf.at[0, pl.ds(0, 16)],
                              acc_sh.at[win_dst.at[0]], sem_s).wait()

    def _win(w, carry):
        pltpu.sync_copy(srcm_ref.at[wid, pl.ds(w * WW, WW)], win_src)
        pltpu.sync_copy(dstm2_ref.at[wid, pl.ds(w * WW * NSUB, WW * NSUB)],
                        win_dst)
        pltpu.sync_copy(e_ref.at[wid, pl.ds(w * WW, WW)], win_e)
        nblk_here = jnp.minimum(WW, NBLK - w * WW)
        pltpu.async_copy(xl_ref.at[win_src.at[0]], xlbuf.at[0], sem_g)

        def _blk(t, c3):
            p = t % 2

            @pl.when(w * WW + t >= 1)
            def _():
                for _ in range(NSUB):
                    _wait_sub()

            @pl.when(t + 1 < nblk_here)
            def _():
                pltpu.async_copy(xl_ref.at[win_src.at[t + 1]],
                                 xlbuf.at[1 - p], sem_g)
            for g in range(BB // 16):
                sl = pl.ds(g * 16, 16)
                dst16 = win_dst[t * NSUB + g, pl.ds(0, 16)]
                e16 = plsc.bitcast(win_e[t, sl], _f32)
                m16 = plsc.load_gather(emax_v, [dst16])
                ee16 = jnp.exp(e16 - m16)
                eebuf[sl] = ee16

                idxm = 512 + (dst16 & 511)
                plsc.store_scatter(eebuf, [idxm], lanef)
                rb = plsc.bitcast(plsc.load_gather(eebuf, [idxm]), _i32)
                winm = rb == lane
                cur = plsc.load_gather(den_v, [dst16])
                plsc.store_scatter(den_v, [dst16], cur + ee16, mask=winm)
                loser = jnp.logical_not(winm)

                @pl.when(jnp.max(loser.astype(_i32)) > 0)
                def _():
                    def _acc(u, c4):
                        cur2 = plsc.load_gather(den_v, [dst16])
                        plsc.store_scatter(den_v, [dst16], cur2 + ee16,
                                           mask=(lane == u) & loser)
                        return c4
                    lax.fori_loop(0, 16, _acc, 0)

            pltpu.make_async_copy(xl_ref.at[win_src.at[t]], xlbuf.at[p],
                                  sem_g).wait()

            for g in range(NSUB):
                def _edge(u, c4):
                    for i in range(4):
                        j = g * 16 + u * 4 + i
                        spl = plsc.load_gather(eebuf,
                                               [jnp.full((16,), j, _i32)])
                        for k in range(H // 16):
                            sl2 = pl.ds(k * 16, 16)
                            xlbuf[p, j, sl2] = xlbuf[p, j, sl2] * spl
                    return c4
                lax.fori_loop(0, 4, _edge, 0)
                pltpu.async_copy(xlbuf.at[p, pl.ds(g * 16, 16)],
                                 acc_sh.at[win_dst.at[t * NSUB + g]],
                                 sem_s, add=True)
            return c3
        lax.fori_loop(0, nblk_here, _blk, 0)
        return carry
    lax.fori_loop(0, (NBLK + WW - 1) // WW, _win, 0)
    for _ in range(NSUB):
        _wait_sub()
    pltpu.sync_copy(den_v, denp_ref.at[wid])
    plsc.subcore_barrier()

    for i in range(RPT // BB):
        sl = pl.ds(base + i * BB, BB)
        pltpu.sync_copy(acc_sh.at[sl], acc_ref.at[c, sl])


def _pass_b(e, pmax, xl, srcm, dstm2):
    mesh = plsc.VectorSubcoreMesh(core_axis_name="c", subcore_axis_name="s")
    f = pl.kernel(
        _pass_b_body,
        out_type=(jax.ShapeDtypeStruct((NC, NPAD, H), _f32),
                  jax.ShapeDtypeStruct((NW, NPAD), _f32)),
        mesh=mesh,
        compiler_params=pltpu.CompilerParams(needs_layout_passes=False),
        scratch_types=[
            pltpu.VMEM((8, BB), _i32),
            pltpu.VMEM((8 * (BB // 16), 16), _i32),
            pltpu.VMEM((8, BB), _i32),
            pltpu.VMEM((NPAD,), _f32),
            pltpu.VMEM((NPAD,), _f32),
            pltpu.VMEM((1024,), _f32),
            pltpu.VMEM((2, BB, H), _f32),
            pltpu.SemaphoreType.DMA,
            pltpu.SemaphoreType.DMA,
            pltpu.VMEM_SHARED((NPAD, H), _f32),
        ],
    )
    return f(e, pmax, xl, srcm, dstm2)


def _mm0_body(x_ref, wl_ref, wr_ref, xl_ref, xr_ref):
    h = x_ref[...]
    xl_ref[...] = jnp.dot(h, wl_ref[...], preferred_element_type=_f32)
    xr_ref[...] = jnp.dot(h, wr_ref[...], preferred_element_type=_f32)


def _mm0(x, wl, wr):
    return pl.pallas_call(
        _mm0_body,
        grid=(NTB,),
        in_specs=[
            pl.BlockSpec((TB, H), lambda i: (i, 0)),
            pl.BlockSpec((H, H), lambda i: (0, 0)),
            pl.BlockSpec((H, H), lambda i: (0, 0)),
        ],
        out_specs=[
            pl.BlockSpec((TB, H), lambda i: (i, 0)),
            pl.BlockSpec((TB, H), lambda i: (i, 0)),
        ],
        out_shape=[
            jax.ShapeDtypeStruct((NPAD, H), _f32),
            jax.ShapeDtypeStruct((NPAD, H), _f32),
        ],
    )(x, wl, wr)


def _combine_h(acc_blk, denp_blk, bias_row):
    den = lax.dot_general(denp_blk, jnp.ones((NW, 1), _f32),
                          (((0,), (0,)), ((), ())),
                          preferred_element_type=_f32)
    return jnp.maximum((acc_blk[0] + acc_blk[1]) / (den + 1e-16)
                       + bias_row, 0.0)


def _comb_body(acc_ref, denp_ref, bias_ref, wl_ref, wr_ref,
               xl_ref, xr_ref):
    h = _combine_h(acc_ref[...], denp_ref[...], bias_ref[...])
    xl_ref[...] = jnp.dot(h, wl_ref[...], preferred_element_type=_f32)
    xr_ref[...] = jnp.dot(h, wr_ref[...], preferred_element_type=_f32)


def _combine_mm(acc, denp, bias_row, wl, wr):
    return pl.pallas_call(
        _comb_body,
        grid=(NTB,),
        in_specs=[
            pl.BlockSpec((NC, TB, H), lambda i: (0, i, 0)),
            pl.BlockSpec((NW, TB), lambda i: (0, i)),
            pl.BlockSpec((1, H), lambda i: (0, 0)),
            pl.BlockSpec((H, H), lambda i: (0, 0)),
            pl.BlockSpec((H, H), lambda i: (0, 0)),
        ],
        out_specs=[
            pl.BlockSpec((TB, H), lambda i: (i, 0)),
            pl.BlockSpec((TB, H), lambda i: (i, 0)),
        ],
        out_shape=[
            jax.ShapeDtypeStruct((NPAD, H), _f32),
            jax.ShapeDtypeStruct((NPAD, H), _f32),
        ],
    )(acc, denp, bias_row, wl, wr)


def _fin_body(acc_ref, denp_ref, bias_ref, batch_ref, w1_ref, b1_ref,
              w2_ref, b2_ref, out_ref, pooled_s, counts_s):
    i = pl.program_id(0)

    @pl.when(i == 0)
    def _():
        pooled_s[...] = jnp.zeros((G, H), _f32)
        counts_s[...] = jnp.zeros((G, H), _f32)

    h = _combine_h(acc_ref[...], denp_ref[...], bias_ref[...])
    bt = batch_ref[0]
    gi = lax.broadcasted_iota(_i32, (G, TB), 0).astype(_f32)
    onehot = jnp.where(bt == gi, 1.0, 0.0)
    pooled_s[...] += jnp.dot(onehot, h, preferred_element_type=_f32)
    cnt = jnp.sum(onehot, axis=1, keepdims=True)
    counts_s[...] += jnp.broadcast_to(cnt, (G, H))

    @pl.when(i == NTB - 1)
    def _():
        pooled = pooled_s[...] / jnp.maximum(counts_s[...], 1.0)
        t = jnp.dot(pooled, w1_ref[...], preferred_element_type=_f32)
        t = jnp.maximum(t + b1_ref[...], 0.0)
        out_ref[...] = (jnp.dot(t, w2_ref[...], preferred_element_type=_f32)
                        + b2_ref[...])


def _final(acc, denp, bias_row, batch3, fc1_W, fc1_b, fc2_W, fc2_b):
    return pl.pallas_call(
        _fin_body,
        grid=(NTB,),
        in_specs=[
            pl.BlockSpec((NC, TB, H), lambda i: (0, i, 0)),
            pl.BlockSpec((NW, TB), lambda i: (0, i)),
            pl.BlockSpec((1, H), lambda i: (0, 0)),
            pl.BlockSpec((1, 1, TB), lambda i: (i, 0, 0)),
            pl.BlockSpec((H, H), lambda i: (0, 0)),
            pl.BlockSpec((1, H), lambda i: (0, 0)),
            pl.BlockSpec((H, H), lambda i: (0, 0)),
            pl.BlockSpec((1, H), lambda i: (0, 0)),
        ],
        out_specs=pl.BlockSpec((G, H), lambda i: (0, 0)),
        out_shape=jax.ShapeDtypeStruct((G, H), _f32),
        scratch_shapes=[
            pltpu.VMEM((G, H), _f32),
            pltpu.VMEM((G, H), _f32),
        ],
    )(acc, denp, bias_row, batch3, fc1_W, fc1_b, fc2_W, fc2_b)


def kernel(x, edge_index, batch, Wl, Wr, att, bias, fc1_W, fc1_b,
           fc2_W, fc2_b):
    x_p = jnp.pad(x, ((0, NPAD - N), (0, 0)))
    ei = edge_index.astype(_i32).reshape(2, NW, NBLK, BB)
    ei = jnp.pad(ei, ((0, 0), (0, 0), (0, NBLKP - NBLK), (0, 0)))
    srcm, dstm = ei[0], ei[1]
    dstm2 = dstm.reshape(NW, NBLKP * (BB // 16), 16)
    batch_p = jnp.concatenate(
        [batch.astype(_i32), jnp.full((NPAD - N,), G, _i32)]).astype(_f32)
    batch3 = batch_p.reshape(NTB, 1, TB)

    xl, xr = _mm0(x_p, Wl[0], Wr[0])
    acc = denp = None
    for l in range(L):
        if l > 0:
            xl, xr = _combine_mm(acc, denp, bias[l - 1][None],
                                 Wl[l], Wr[l])
        e, pmax = _pass_a(xl, xr, srcm, dstm, att[l])
        acc, denp = _pass_b(e, pmax, xl, srcm, dstm2)
    return _final(acc, denp, bias[L - 1][None], batch3,
                  fc1_W, fc1_b[None], fc2_W, fc2_b[None])


# speedup vs baseline: 1.2909x; 1.2862x over previous
import functools

import jax
import jax.numpy as jnp
from jax import lax
from jax.experimental import pallas as pl
from jax.experimental.pallas import tpu as pltpu
from jax.experimental.pallas import tpu_sc as plsc

N = 10000
NPAD = 10240
E = 320000
H = 128
G = 64
L = 4

NC = 2
NS = 16
NW = NC * NS
EPW = E // NW
BB = 80
NBLK = EPW // BB
NBLKP = 513
NBLKV = 128
ACCW = 144
RPT = NPAD // NS
TB = 512
NTB = NPAD // TB

_f32 = jnp.float32
_i32 = jnp.int32


def _widx():
    return lax.axis_index("s") * NC + lax.axis_index("c")


def _pass_a_body(xl_ref, xr_ref, srcm_ref, dstm_ref, att_ref,
                 e_ref, pmax_ref,
                 src_v, dst_v, att_v, emax_v, e_v, abuf, tmpb, xlbuf, xrbuf,
                 sem_g):
    wid = _widx()
    pltpu.sync_copy(srcm_ref.at[wid, pl.ds(0, NBLKV)], src_v)
    pltpu.sync_copy(dstm_ref.at[wid, pl.ds(0, NBLKV)], dst_v)
    pltpu.sync_copy(att_ref, att_v)

    def _init(i, carry):
        emax_v[pl.ds(i * 16, 16)] = jnp.full((16,), -1e30, _f32)
        return carry
    lax.fori_loop(0, NPAD // 16, _init, 0)

    lane = lax.iota(_i32, 16)
    att16 = [att_v[pl.ds(k * 16, 16)] for k in range(H // 16)]

    pltpu.async_copy(xl_ref.at[src_v.at[0]], xlbuf.at[0], sem_g)
    pltpu.async_copy(xr_ref.at[dst_v.at[0]], xrbuf.at[0], sem_g)

    def _blk(b, carry):
        p = b % 2

        @pl.when(b + 1 < NBLK)
        def _():
            pltpu.async_copy(xl_ref.at[src_v.at[b + 1]], xlbuf.at[1 - p],
                             sem_g)
            pltpu.async_copy(xr_ref.at[dst_v.at[b + 1]], xrbuf.at[1 - p],
                             sem_g)
        pltpu.make_async_copy(xl_ref.at[src_v.at[b]], xlbuf.at[p],
                              sem_g).wait()
        pltpu.make_async_copy(xr_ref.at[dst_v.at[b]], xrbuf.at[p],
                              sem_g).wait()
        for g in range(BB // 16):
            def _edge(u, c2):
                for i in range(4):
                    j = g * 16 + u * 4 + i
                    acc = jnp.zeros((16,), _f32)
                    for k in range(H // 16):
                        sl = pl.ds(k * 16, 16)
                        z = xlbuf[p, j, sl] + xrbuf[p, j, sl]
                        zl = jnp.where(z > 0, z, 0.2 * z)
                        acc = acc + zl * att16[k]
                    plsc.store_scatter(abuf, [lane * 16 + (u * 4 + i)], acc)
                return c2
            lax.fori_loop(0, 4, _edge, 0)
            e16 = abuf[pl.ds(0, 16)]
            for l in range(1, 16):
                e16 = e16 + abuf[pl.ds(l * 16, 16)]
            e_v[b, pl.ds(g * 16, 16)] = plsc.bitcast(e16, _i32)
            dst16 = dst_v[b, pl.ds(g * 16, 16)]

            idxm = dst16 & (2048 - 1)
            plsc.store_scatter(tmpb, [idxm], lane)
            rb = plsc.load_gather(tmpb, [idxm])
            winm = rb == lane
            cur = plsc.load_gather(emax_v, [dst16])
            plsc.store_scatter(emax_v, [dst16], jnp.maximum(cur, e16),
                               mask=winm)
            loser = jnp.logical_not(winm)

            @pl.when(jnp.max(loser.astype(_i32)) > 0)
            def _():
                def _mx(t, c2):
                    cur2 = plsc.load_gather(emax_v, [dst16])
                    need = (e16 > cur2) & (lane == t) & loser
                    plsc.store_scatter(emax_v, [dst16], e16, mask=need)
                    return c2
                lax.fori_loop(0, 16, _mx, 0)
        return carry
    lax.fori_loop(0, NBLK, _blk, 0)

    pltpu.sync_copy(e_v, e_ref.at[wid, pl.ds(0, NBLKV)])
    pltpu.sync_copy(emax_v, pmax_ref.at[wid])


def _pass_a(xl, xr, srcm, dstm, att_l):
    mesh = plsc.VectorSubcoreMesh(core_axis_name="c", subcore_axis_name="s")
    f = pl.kernel(
        _pass_a_body,
        out_type=(jax.ShapeDtypeStruct((NW, NBLKP, BB), _i32),
                  jax.ShapeDtypeStruct((NW, NPAD), _f32)),
        mesh=mesh,
        compiler_params=pltpu.CompilerParams(needs_layout_passes=False),
        scratch_types=[
            pltpu.VMEM((NBLKV, BB), _i32),
            pltpu.VMEM((NBLKV, BB), _i32),
            pltpu.VMEM((H,), _f32),
            pltpu.VMEM((NPAD,), _f32),
            pltpu.VMEM((NBLKV, BB), _i32),
            pltpu.VMEM((256,), _f32),
            pltpu.VMEM((2048,), _i32),
            pltpu.VMEM((2, BB, H), _f32),
            pltpu.VMEM((2, BB, H), _f32),
            pltpu.SemaphoreType.DMA,
        ],
    )
    return f(xl, xr, srcm, dstm, att_l)


def _pass_b_body(e_ref, pmax_ref, xl_ref, srcm_ref, dstm2_ref,
                 acc_ref, denp_ref,
                 win_src, win_dst, ebuf, emax_v, den_v, eebuf, xlbuf,
                 sem_g, sem_s, acc_sh, emax_sh):
    c = lax.axis_index("c")
    s = lax.axis_index("s")
    wid = s * NC + c

    base0 = s * RPT
    for r in range(2):
        for p16 in range(16):
            pltpu.async_copy(pmax_ref.at[r * 16 + p16, pl.ds(base0, RPT)],
                             den_v.at[pl.ds(p16 * RPT, RPT)], sem_g)
        for p16 in range(16):
            pltpu.make_async_copy(
                pmax_ref.at[r * 16 + p16, pl.ds(base0, RPT)],
                den_v.at[pl.ds(p16 * RPT, RPT)], sem_g).wait()

        def _red(i, c2):
            m16 = den_v[pl.ds(i * 16, 16)]
            for p16 in range(1, 16):
                m16 = jnp.maximum(m16, den_v[pl.ds(p16 * RPT + i * 16, 16)])
            sl = pl.ds(base0 + i * 16, 16)
            if r == 0:
                emax_v[sl] = m16
            else:
                emax_v[sl] = jnp.maximum(emax_v[sl], m16)
            return c2
        lax.fori_loop(0, RPT // 16, _red, 0)
    pltpu.sync_copy(emax_v.at[pl.ds(base0, RPT)],
                    emax_sh.at[pl.ds(base0, RPT)])

    def _dz(i, carry):
        den_v[pl.ds(i * 16, 16)] = jnp.zeros((16,), _f32)
        return carry
    lax.fori_loop(0, NPAD // 16, _dz, 0)

    def _zr(r, carry):
        for j in range(H // 16):
            xlbuf[0, r, pl.ds(j * 16, 16)] = jnp.zeros((16,), _f32)
        return carry
    lax.fori_loop(0, BB, _zr, 0)
    base = s * RPT
    for i in range(RPT // BB):
        pltpu.sync_copy(xlbuf.at[0], acc_sh.at[pl.ds(base + i * BB, BB)])
    plsc.subcore_barrier()
    pltpu.sync_copy(emax_sh, emax_v)

    lane = lax.iota(_i32, 16)
    lanef = plsc.bitcast(lane, _f32)
    WW = 8
    NSUB = BB // 16

    def _wait_sub():
        pltpu.make_async_copy(xlbuf.at[0, pl.ds(0, 16)],
                              acc_sh.at[win_dst.at[0]], sem_s).wait()

    def _win(w, carry):
        pltpu.sync_copy(srcm_ref.at[wid, pl.ds(w * WW, WW)], win_src)
        pltpu.sync_copy(dstm2_ref.at[wid, pl.ds(w * WW * NSUB, WW * NSUB)],
                        win_dst)
        nblk_here = jnp.minimum(WW, NBLK - w * WW)
        pltpu.async_copy(xl_ref.at[win_src.at[0]], xlbuf.at[0], sem_g)
        pltpu.async_copy(e_ref.at[wid, w * WW], ebuf.at[0], sem_g)

        def _blk(t, c3):
            p = t % 2

            @pl.when(w * WW + t >= 1)
            def _():
                for _ in range(NSUB):
                    _wait_sub()

            @pl.when(t + 1 < nblk_here)
            def _():
                pltpu.async_copy(xl_ref.at[win_src.at[t + 1]],
                                 xlbuf.at[1 - p], sem_g)
                pltpu.async_copy(e_ref.at[wid, w * WW + t + 1],
                                 ebuf.at[1 - p], sem_g)
            pltpu.make_async_copy(e_ref.at[wid, w * WW + t], ebuf.at[p],
                                  sem_g).wait()
            for g in range(BB // 16):
                sl = pl.ds(g * 16, 16)
                dst16 = win_dst[t * NSUB + g, pl.ds(0, 16)]
                e16 = plsc.bitcast(ebuf[p, sl], _f32)
                m16 = plsc.load_gather(emax_v, [dst16])
                ee16 = jnp.exp(e16 - m16)
                eebuf[sl] = ee16

                idxm = 80 + (dst16 & 511)
                plsc.store_scatter(eebuf, [idxm], lanef)
                rb = plsc.bitcast(plsc.load_gather(eebuf, [idxm]), _i32)
                winm = rb == lane
                cur = plsc.load_gather(den_v, [dst16])
                plsc.store_scatter(den_v, [dst16], cur + ee16, mask=winm)
                loser = jnp.logical_not(winm)

                @pl.when(jnp.max(loser.astype(_i32)) > 0)
                def _():
                    def _acc(u, c4):
                        cur2 = plsc.load_gather(den_v, [dst16])
                        plsc.store_scatter(den_v, [dst16], cur2 + ee16,
                                           mask=(lane == u) & loser)
                        return c4
                    lax.fori_loop(0, 16, _acc, 0)

            pltpu.make_async_copy(xl_ref.at[win_src.at[t]], xlbuf.at[p],
                                  sem_g).wait()

            for g in range(NSUB):
                def _edge(u, c4):
                    for i in range(4):
                        j = g * 16 + u * 4 + i
                        spl = plsc.load_gather(eebuf,
                                               [jnp.full((16,), j, _i32)])
                        for k in range(H // 16):
                            sl2 = pl.ds(k * 16, 16)
                            xlbuf[p, j, sl2] = xlbuf[p, j, sl2] * spl
                    return c4
                lax.fori_loop(0, 4, _edge, 0)
                pltpu.async_copy(xlbuf.at[p, pl.ds(g * 16, 16)],
                                 acc_sh.at[win_dst.at[t * NSUB + g]],
                                 sem_s, add=True)
            return c3
        lax.fori_loop(0, nblk_here, _blk, 0)
        return carry
    lax.fori_loop(0, (NBLK + WW - 1) // WW, _win, 0)
    for _ in range(NSUB):
        _wait_sub()
    pltpu.sync_copy(den_v, denp_ref.at[wid])
    plsc.subcore_barrier()

    for i in range(RPT // BB):
        sl = pl.ds(base + i * BB, BB)
        pltpu.sync_copy(acc_sh.at[sl], acc_ref.at[c, sl])


def _pass_b(e, pmax, xl, srcm, dstm2):
    mesh = plsc.VectorSubcoreMesh(core_axis_name="c", subcore_axis_name="s")
    f = pl.kernel(
        _pass_b_body,
        out_type=(jax.ShapeDtypeStruct((NC, NPAD, H), _f32),
                  jax.ShapeDtypeStruct((NW, NPAD), _f32)),
        mesh=mesh,
        compiler_params=pltpu.CompilerParams(needs_layout_passes=False),
        scratch_types=[
            pltpu.VMEM((8, BB), _i32),
            pltpu.VMEM((8 * (BB // 16), 16), _i32),
            pltpu.VMEM((2, BB), _i32),
            pltpu.VMEM((NPAD,), _f32),
            pltpu.VMEM((NPAD,), _f32),
            pltpu.VMEM((592,), _f32),
            pltpu.VMEM((2, BB, H), _f32),
            pltpu.SemaphoreType.DMA,
            pltpu.SemaphoreType.DMA,
            pltpu.VMEM_SHARED((NPAD, H), _f32),
            pltpu.VMEM_SHARED((NPAD,), _f32),
        ],
    )
    return f(e, pmax, xl, srcm, dstm2)


def _mm0_body(x_ref, wl_ref, wr_ref, xl_ref, xr_ref):
    h = x_ref[...]
    xl_ref[...] = jnp.dot(h, wl_ref[...], preferred_element_type=_f32)
    xr_ref[...] = jnp.dot(h, wr_ref[...], preferred_element_type=_f32)


def _mm0(x, wl, wr):
    return pl.pallas_call(
        _mm0_body,
        grid=(NTB,),
        in_specs=[
            pl.BlockSpec((TB, H), lambda i: (i, 0)),
            pl.BlockSpec((H, H), lambda i: (0, 0)),
            pl.BlockSpec((H, H), lambda i: (0, 0)),
        ],
        out_specs=[
            pl.BlockSpec((TB, H), lambda i: (i, 0)),
            pl.BlockSpec((TB, H), lambda i: (i, 0)),
        ],
        out_shape=[
            jax.ShapeDtypeStruct((NPAD, H), _f32),
            jax.ShapeDtypeStruct((NPAD, H), _f32),
        ],
    )(x, wl, wr)


def _combine_h(acc_blk, denp_blk, bias_row):
    den = lax.dot_general(denp_blk, jnp.ones((NW, 1), _f32),
                          (((0,), (0,)), ((), ())),
                          preferred_element_type=_f32)
    return jnp.maximum((acc_blk[0] + acc_blk[1]) / (den + 1e-16)
                       + bias_row, 0.0)


def _comb_body(acc_ref, denp_ref, bias_ref, wl_ref, wr_ref,
               xl_ref, xr_ref):
    h = _combine_h(acc_ref[...], denp_ref[...], bias_ref[...])
    xl_ref[...] = jnp.dot(h, wl_ref[...], preferred_element_type=_f32)
    xr_ref[...] = jnp.dot(h, wr_ref[...], preferred_element_type=_f32)


def _combine_mm(acc, denp, bias_row, wl, wr):
    return pl.pallas_call(
        _comb_body,
        grid=(NTB,),
        in_specs=[
            pl.BlockSpec((NC, TB, H), lambda i: (0, i, 0)),
            pl.BlockSpec((NW, TB), lambda i: (0, i)),
            pl.BlockSpec((1, H), lambda i: (0, 0)),
            pl.BlockSpec((H, H), lambda i: (0, 0)),
            pl.BlockSpec((H, H), lambda i: (0, 0)),
        ],
        out_specs=[
            pl.BlockSpec((TB, H), lambda i: (i, 0)),
            pl.BlockSpec((TB, H), lambda i: (i, 0)),
        ],
        out_shape=[
            jax.ShapeDtypeStruct((NPAD, H), _f32),
            jax.ShapeDtypeStruct((NPAD, H), _f32),
        ],
    )(acc, denp, bias_row, wl, wr)


def _fin_body(acc_ref, denp_ref, bias_ref, batch_ref, w1_ref, b1_ref,
              w2_ref, b2_ref, out_ref, pooled_s, counts_s):
    i = pl.program_id(0)

    @pl.when(i == 0)
    def _():
        pooled_s[...] = jnp.zeros((G, H), _f32)
        counts_s[...] = jnp.zeros((G, H), _f32)

    h = _combine_h(acc_ref[...], denp_ref[...], bias_ref[...])
    bt = batch_ref[0]
    gi = lax.broadcasted_iota(_i32, (G, TB), 0).astype(_f32)
    onehot = jnp.where(bt == gi, 1.0, 0.0)
    pooled_s[...] += jnp.dot(onehot, h, preferred_element_type=_f32)
    cnt = jnp.sum(onehot, axis=1, keepdims=True)
    counts_s[...] += jnp.broadcast_to(cnt, (G, H))

    @pl.when(i == NTB - 1)
    def _():
        pooled = pooled_s[...] / jnp.maximum(counts_s[...], 1.0)
        t = jnp.dot(pooled, w1_ref[...], preferred_element_type=_f32)
        t = jnp.maximum(t + b1_ref[...], 0.0)
        out_ref[...] = (jnp.dot(t, w2_ref[...], preferred_element_type=_f32)
                        + b2_ref[...])


def _final(acc, denp, bias_row, batch3, fc1_W, fc1_b, fc2_W, fc2_b):
    return pl.pallas_call(
        _fin_body,
        grid=(NTB,),
        in_specs=[
            pl.BlockSpec((NC, TB, H), lambda i: (0, i, 0)),
            pl.BlockSpec((NW, TB), lambda i: (0, i)),
            pl.BlockSpec((1, H), lambda i: (0, 0)),
            pl.BlockSpec((1, 1, TB), lambda i: (i, 0, 0)),
            pl.BlockSpec((H, H), lambda i: (0, 0)),
            pl.BlockSpec((1, H), lambda i: (0, 0)),
            pl.BlockSpec((H, H), lambda i: (0, 0)),
            pl.BlockSpec((1, H), lambda i: (0, 0)),
        ],
        out_specs=pl.BlockSpec((G, H), lambda i: (0, 0)),
        out_shape=jax.ShapeDtypeStruct((G, H), _f32),
        scratch_shapes=[
            pltpu.VMEM((G, H), _f32),
            pltpu.VMEM((G, H), _f32),
        ],
    )(acc, denp, bias_row, batch3, fc1_W, fc1_b, fc2_W, fc2_b)


def kernel(x, edge_index, batch, Wl, Wr, att, bias, fc1_W, fc1_b,
           fc2_W, fc2_b):
    x_p = jnp.pad(x, ((0, NPAD - N), (0, 0)))
    ei = edge_index.astype(_i32).reshape(2, NW, NBLK, BB)
    ei = jnp.pad(ei, ((0, 0), (0, 0), (0, NBLKP - NBLK), (0, 0)))
    srcm, dstm = ei[0], ei[1]
    dstm2 = dstm.reshape(NW, NBLKP * (BB // 16), 16)
    batch_p = jnp.concatenate(
        [batch.astype(_i32), jnp.full((NPAD - N,), G, _i32)]).astype(_f32)
    batch3 = batch_p.reshape(NTB, 1, TB)

    xl, xr = _mm0(x_p, Wl[0], Wr[0])
    acc = denp = None
    for l in range(L):
        if l > 0:
            xl, xr = _combine_mm(acc, denp, bias[l - 1][None],
                                 Wl[l], Wr[l])
        e, pmax = _pass_a(xl, xr, srcm, dstm, att[l])
        acc, denp = _pass_b(e, pmax, xl, srcm, dstm2)
    return _final(acc, denp, bias[L - 1][None], batch3,
                  fc1_W, fc1_b[None], fc2_W, fc2_b[None])
